# baseline (speedup 1.0000x reference)
import functools
import math

import jax
import jax.numpy as jnp
from jax import lax
from jax.experimental import pallas as pl
from jax.experimental.pallas import tpu as pltpu
from jax.experimental.pallas import tpu_sc as plsc

D_MODEL = 2048
SCALE = math.sqrt(D_MODEL)
LANES = 16
NUM_CORES = 2
NUM_SUBCORES = 16
NUM_WORKERS = NUM_CORES * NUM_SUBCORES

CHUNK = 8
NBUF = 4
AHEAD = 2
WBACK = NBUF - AHEAD


def _build_kernel(B):
    b_per_w = B // NUM_WORKERS
    n_chunks = b_per_w // CHUNK
    mesh = plsc.VectorSubcoreMesh(core_axis_name="c", subcore_axis_name="s")

    @functools.partial(
        pl.kernel,
        mesh=mesh,
        out_type=jax.ShapeDtypeStruct((B, D_MODEL), jnp.float32),
        scratch_types=[
            pltpu.VMEM((b_per_w,), jnp.int32),
            pltpu.VMEM((NBUF, CHUNK, D_MODEL), jnp.float32),
            pltpu.SemaphoreType.DMA,
            pltpu.SemaphoreType.DMA,
        ],
    )
    def k(lut_hbm, idx_hbm, out_hbm, idx_v, rows_v, gsem, ssem):
        wid = lax.axis_index("s") * NUM_CORES + lax.axis_index("c")
        base = wid * b_per_w
        pltpu.sync_copy(idx_hbm.at[pl.ds(base, b_per_w)], idx_v)

        def start_gather(cg, buf):
            pltpu.async_copy(
                lut_hbm.at[idx_v.at[pl.ds(cg * CHUNK, CHUNK)]],
                rows_v.at[buf],
                gsem,
            )

        def wait_scatter(buf):
            pltpu.make_async_copy(
                rows_v.at[buf], out_hbm.at[pl.ds(base, CHUNK)], ssem
            ).wait()

        def chunk_body(cg, b, static):
            nb = (b + AHEAD) % NBUF

            def free_and_refill():
                def _w():
                    wait_scatter(nb)

                def _g():
                    start_gather(cg + AHEAD, nb)

                if static:
                    if cg >= WBACK:
                        _w()
                    if cg + AHEAD < n_chunks:
                        _g()
                else:
                    pl.when(cg >= WBACK)(_w)
                    pl.when(cg + AHEAD < n_chunks)(_g)

            free_and_refill()

            pltpu.make_async_copy(
                lut_hbm.at[pl.ds(0, CHUNK)], rows_v.at[b], gsem
            ).wait()

            for i in range(CHUNK):
                for j in range(D_MODEL // LANES):
                    sl = pl.ds(j * LANES, LANES)
                    rows_v[b, i, sl] = rows_v[b, i, sl] * SCALE

            pltpu.async_copy(
                rows_v.at[b], out_hbm.at[pl.ds(base + cg * CHUNK, CHUNK)], ssem
            )

        for p in range(AHEAD):
            start_gather(p, p)

        n_main = (n_chunks // NBUF) * NBUF

        @pl.loop(0, n_main, step=NBUF)
        def _chunk(g):
            for b in range(NBUF):
                chunk_body(g + b, b, static=False)

        for cg in range(n_main, n_chunks):
            chunk_body(cg, cg % NBUF, static=True)

        for cg in range(n_chunks - WBACK, n_chunks):
            wait_scatter(cg % NBUF)

    return k


def kernel(x, lut):
    B = x.size
    idx = x.reshape(B).astype(jnp.int32)
    out = _build_kernel(B)(lut, idx)
    return out.reshape(x.shape + (D_MODEL,))


# speedup vs baseline: 1.3372x; 1.3372x over previous
import functools
import math

import jax
import jax.numpy as jnp
from jax import lax
from jax.experimental import pallas as pl
from jax.experimental.pallas import tpu as pltpu
from jax.experimental.pallas import tpu_sc as plsc

D_MODEL = 2048
SCALE = math.sqrt(D_MODEL)
LANES = 16
NUM_CORES = 2
NUM_SUBCORES = 16
NUM_WORKERS = NUM_CORES * NUM_SUBCORES

CHUNK = 8
NBUF = 4
AHEAD = 2
WBACK = NBUF - AHEAD


def _build_kernel(B):
    b_per_w = B // NUM_WORKERS
    n_chunks = b_per_w // CHUNK
    mesh = plsc.VectorSubcoreMesh(core_axis_name="c", subcore_axis_name="s")

    @functools.partial(
        pl.kernel,
        mesh=mesh,
        out_type=jax.ShapeDtypeStruct((B, D_MODEL), jnp.float32),
        scratch_types=[
            pltpu.VMEM((b_per_w,), jnp.int32),
            pltpu.VMEM((NBUF, CHUNK, D_MODEL), jnp.float32),
            pltpu.SemaphoreType.DMA,
            pltpu.SemaphoreType.DMA,
        ],
    )
    def k(lut_hbm, idx_hbm, out_hbm, idx_v, rows_v, gsem, ssem):
        wid = lax.axis_index("s") * NUM_CORES + lax.axis_index("c")
        base = wid * b_per_w
        pltpu.sync_copy(idx_hbm.at[pl.ds(base, b_per_w)], idx_v)

        def start_gather(cg, buf):
            pltpu.async_copy(
                lut_hbm.at[idx_v.at[pl.ds(cg * CHUNK, CHUNK)]],
                rows_v.at[buf],
                gsem,
            )

        def wait_scatter(buf):
            pltpu.make_async_copy(
                rows_v.at[buf], out_hbm.at[pl.ds(base, CHUNK)], ssem
            ).wait()

        def chunk_body(cg, b, static):
            nb = (b + AHEAD) % NBUF

            def free_and_refill():
                def _w():
                    wait_scatter(nb)

                def _g():
                    start_gather(cg + AHEAD, nb)

                if static:
                    if cg >= WBACK:
                        _w()
                    if cg + AHEAD < n_chunks:
                        _g()
                else:
                    pl.when(cg >= WBACK)(_w)
                    pl.when(cg + AHEAD < n_chunks)(_g)

            free_and_refill()

            pltpu.make_async_copy(
                lut_hbm.at[pl.ds(0, CHUNK)], rows_v.at[b], gsem
            ).wait()

            @pl.loop(0, CHUNK)
            def _row(i):
                for j in range(D_MODEL // LANES):
                    sl = pl.ds(j * LANES, LANES)
                    rows_v[b, i, sl] = rows_v[b, i, sl] * SCALE

            pltpu.async_copy(
                rows_v.at[b], out_hbm.at[pl.ds(base + cg * CHUNK, CHUNK)], ssem
            )

        for p in range(AHEAD):
            start_gather(p, p)

        n_main = (n_chunks // NBUF) * NBUF

        @pl.loop(0, n_main, step=NBUF)
        def _chunk(g):
            for b in range(NBUF):
                chunk_body(g + b, b, static=False)

        for cg in range(n_main, n_chunks):
            chunk_body(cg, cg % NBUF, static=True)

        for cg in range(n_chunks - WBACK, n_chunks):
            wait_scatter(cg % NBUF)

    return k


def kernel(x, lut):
    B = x.size
    idx = x.reshape(B).astype(jnp.int32)
    out = _build_kernel(B)(lut, idx)
    return out.reshape(x.shape + (D_MODEL,))
